# baseline (speedup 1.0000x reference)
import functools

import jax
import jax.numpy as jnp
from jax import lax
from jax.experimental import pallas as pl
from jax.experimental.pallas import tpu as pltpu
from jax.experimental.pallas import tpu_sc as plsc

VOCAB = 1000
EMBED = 64
BATCH = 16384

NC = 2
NS = 16
NW = NC * NS
B_PER_W = BATCH // NW
CHUNK = 128
NCHUNK = B_PER_W // CHUNK
LANES = 16
EB = EMBED // 8
BB = BATCH // CHUNK

_MESH = plsc.VectorSubcoreMesh(core_axis_name="c", subcore_axis_name="s")


@functools.partial(
    pl.kernel,
    mesh=_MESH,
    out_type=jax.ShapeDtypeStruct((EB, BB, 8, CHUNK), jnp.float32),
    scratch_types=[
        pltpu.VMEM((B_PER_W,), jnp.int32),
        pltpu.VMEM((B_PER_W,), jnp.float32),
        pltpu.VMEM((B_PER_W, EMBED), jnp.float32),
        pltpu.VMEM((NCHUNK, EB, 8, CHUNK), jnp.float32),
        pltpu.SemaphoreType.DMA,
        pltpu.SemaphoreType.DMA,
    ],
    compiler_params=pltpu.CompilerParams(
        use_tc_tiling_on_sc=False, needs_layout_passes=False
    ),
)
def _embed_lookup(idx_hbm, table_hbm, out_hbm, idx_v, mask_v, rows_v, blk_v, gsem, osem):
    wid = lax.axis_index("s") * NC + lax.axis_index("c")
    base = wid * B_PER_W
    pltpu.sync_copy(idx_hbm.at[pl.ds(base, B_PER_W)], idx_v)
    gathers = []
    for j in range(NCHUNK):
        for i in range(CHUNK // LANES):
            v = idx_v[pl.ds(j * CHUNK + i * LANES, LANES)]
            mask_v[pl.ds(j * CHUNK + i * LANES, LANES)] = jnp.where(
                v == 0, 0.0, 1.0
            )
        gathers.append(
            pltpu.async_copy(
                table_hbm.at[idx_v.at[pl.ds(j * CHUNK, CHUNK)]],
                rows_v.at[pl.ds(j * CHUNK, CHUNK)],
                gsem,
            )
        )
    lane = lax.broadcasted_iota(jnp.int32, (LANES,), 0)
    writes = []
    for j in range(NCHUNK):
        gathers[j].wait()
        mvecs = [mask_v[pl.ds(j * CHUNK + g * LANES, LANES)] for g in range(8)]
        rowidx = [lane + (j * CHUNK + g * LANES) for g in range(8)]
        for eb in range(EB):

            def body(es, carry, j=j, eb=eb):
                colidx = jnp.full((LANES,), eb * 8, jnp.int32) + es
                for g in range(8):
                    vec = plsc.load_gather(rows_v, [rowidx[g], colidx])
                    blk_v[j, eb, es, pl.ds(g * LANES, LANES)] = vec * mvecs[g]
                return carry

            lax.fori_loop(0, 8, body, jnp.int32(0))
        writes.append(
            pltpu.async_copy(
                blk_v.at[j],
                out_hbm.at[:, wid * NCHUNK + j],
                osem,
            )
        )
    for c in writes:
        c.wait()


def kernel(age, table):
    arr4 = _embed_lookup(age.astype(jnp.int32), table)
    return arr4.transpose(1, 3, 0, 2).reshape(BATCH, EMBED)
